# Initial kernel scaffold; baseline (speedup 1.0000x reference)
import jax
import jax.numpy as jnp
from jax.experimental import pallas as pl


def kernel(x, w1, w2, g1, b1, g2, b2, ws, gs, bs):
    raise NotImplementedError("write your pallas kernel here")


# speedup vs baseline: 1.1224x; 1.1224x over previous
import functools
import math

import jax
import jax.numpy as jnp
from jax import lax
from jax.experimental import pallas as pl
from jax.experimental.pallas import tpu as pltpu

_EPS = 1e-5


def _cparams(vmem_mb=64):
    return pltpu.CompilerParams(
        dimension_semantics=("parallel",),
        vmem_limit_bytes=vmem_mb * 2**20,
    )


def _stats2(y):
    return jnp.concatenate(
        [jnp.sum(y, axis=0, keepdims=True),
         jnp.sum(y * y, axis=0, keepdims=True)], axis=0)


def _bn_affine(st_sum, st_sq, g, b, count):
    mu = st_sum / count
    var = st_sq / count - mu * mu
    scale = g * lax.rsqrt(var + _EPS)
    shift = b - mu * scale
    return scale.reshape(1, -1), shift.reshape(1, -1)


def _stage_a_kernel(x_ref, w1b_ref, w0_ref, w2_ref, y1_ref, st_ref, xpad):
    l, cin = x_ref.shape[1], x_ref.shape[2]
    x = x_ref[0]
    xpad[0:1, :] = jnp.zeros((1, cin), jnp.float32)
    xpad[l + 1:l + 2, :] = jnp.zeros((1, cin), jnp.float32)
    xpad[1:l + 1, :] = x.astype(jnp.float32)
    p = jnp.dot(x, w1b_ref[...], preferred_element_type=jnp.float32)
    cout = w0_ref.shape[1]
    y1 = p[:, :cout]
    sc = p[:, cout:]
    y1 = y1 + jnp.dot(xpad[0:l, :].astype(jnp.bfloat16), w0_ref[...],
                      preferred_element_type=jnp.float32)
    y1 = y1 + jnp.dot(xpad[2:l + 2, :].astype(jnp.bfloat16), w2_ref[...],
                      preferred_element_type=jnp.float32)
    y1_ref[0] = y1.astype(jnp.bfloat16)
    st_ref[0, 0:2, :] = _stats2(y1)
    st_ref[0, 2:4, :] = _stats2(sc)
    st_ref[0, 4:8, :] = jnp.zeros((4, cout), jnp.float32)


def _stage_b_kernel(y1_ref, s1_ref, h1_ref, w0_ref, w1_ref, w2_ref,
                    z_ref, st_ref, apad):
    l, c = y1_ref.shape[1], y1_ref.shape[2]
    a = jnp.maximum(
        y1_ref[0].astype(jnp.float32) * s1_ref[...] + h1_ref[...], 0.0)
    apad[0:1, :] = jnp.zeros((1, c), jnp.float32)
    apad[l + 1:l + 2, :] = jnp.zeros((1, c), jnp.float32)
    apad[1:l + 1, :] = a
    z = jnp.dot(a.astype(jnp.bfloat16), w1_ref[...],
                preferred_element_type=jnp.float32)
    z = z + jnp.dot(apad[0:l, :].astype(jnp.bfloat16), w0_ref[...],
                    preferred_element_type=jnp.float32)
    z = z + jnp.dot(apad[2:l + 2, :].astype(jnp.bfloat16), w2_ref[...],
                    preferred_element_type=jnp.float32)
    z_ref[0] = z.astype(jnp.bfloat16)
    st_ref[0, 0:2, :] = _stats2(z)
    st_ref[0, 2:8, :] = jnp.zeros((6, c), jnp.float32)


def _stage_c_kernel(z_ref, x_ref, ws_ref, s2_ref, h2_ref, ss_ref, hs_ref,
                    out_ref):
    sc = jnp.dot(x_ref[0], ws_ref[...], preferred_element_type=jnp.float32)
    z = z_ref[0].astype(jnp.float32) * s2_ref[...] + h2_ref[...]
    s = sc * ss_ref[...] + hs_ref[...]
    out_ref[0] = jnp.maximum(z + s, 0.0)


@jax.jit
def _block(x, w1, w2, g1, b1, g2, b2, ws, gs, bs):
    N, Cin, L = x.shape
    Cout = w1.shape[0]
    count = float(N * L)

    x_cl = jnp.transpose(x, (0, 2, 1)).astype(jnp.bfloat16)
    w1t = jnp.transpose(w1, (2, 1, 0)).astype(jnp.bfloat16)
    w2t = jnp.transpose(w2, (2, 1, 0)).astype(jnp.bfloat16)
    wst = jnp.transpose(ws, (2, 1, 0))[0].astype(jnp.bfloat16)
    w1b = jnp.concatenate([w1t[1], wst], axis=1)

    def full(shp):
        n = len(shp)
        return pl.BlockSpec(shp, lambda b: (0,) * n)

    x_spec = pl.BlockSpec((1, L, Cin), lambda b: (b, 0, 0))
    row_spec = pl.BlockSpec((1, L, Cout), lambda b: (b, 0, 0))
    st_spec = pl.BlockSpec((1, 8, Cout), lambda b: (b, 0, 0))
    vec_spec = pl.BlockSpec((1, Cout), lambda b: (0, 0))

    row_bf = jax.ShapeDtypeStruct((N, L, Cout), jnp.bfloat16)
    st_struct = jax.ShapeDtypeStruct((N, 8, Cout), jnp.float32)

    y1, sta = pl.pallas_call(
        _stage_a_kernel,
        grid=(N,),
        in_specs=[x_spec, full((Cin, 2 * Cout)), full((Cin, Cout)),
                  full((Cin, Cout))],
        out_specs=[row_spec, st_spec],
        out_shape=[row_bf, st_struct],
        scratch_shapes=[pltpu.VMEM((L + 2, Cin), jnp.float32)],
        compiler_params=_cparams(),
    )(x_cl, w1b, w1t[0], w1t[2])

    st = jnp.sum(sta, axis=0)
    scale1, shift1 = _bn_affine(st[0], st[1], g1, b1, count)
    scale_s, shift_s = _bn_affine(st[2], st[3], gs, bs, count)

    z, stb = pl.pallas_call(
        _stage_b_kernel,
        grid=(N,),
        in_specs=[row_spec, vec_spec, vec_spec, full((Cout, Cout)),
                  full((Cout, Cout)), full((Cout, Cout))],
        out_specs=[row_spec, st_spec],
        out_shape=[row_bf, st_struct],
        scratch_shapes=[pltpu.VMEM((L + 2, Cout), jnp.float32)],
        compiler_params=_cparams(),
    )(y1, scale1, shift1, w2t[0], w2t[1], w2t[2])

    st2 = jnp.sum(stb, axis=0)
    scale2, shift2 = _bn_affine(st2[0], st2[1], g2, b2, count)

    out_p = pl.pallas_call(
        _stage_c_kernel,
        grid=(N,),
        in_specs=[row_spec, x_spec, full((Cin, Cout)), vec_spec, vec_spec,
                  vec_spec, vec_spec],
        out_specs=row_spec,
        out_shape=jax.ShapeDtypeStruct((N, L, Cout), jnp.float32),
        compiler_params=_cparams(),
    )(z, x_cl, wst, scale2, shift2, scale_s, shift_s)

    return jnp.transpose(out_p, (0, 2, 1))


def kernel(x, w1, w2, g1, b1, g2, b2, ws, gs, bs):
    return _block(x, w1, w2, g1, b1, g2, b2, ws, gs, bs)


# speedup vs baseline: 1.3486x; 1.2015x over previous
import functools
import math

import jax
import jax.numpy as jnp
from jax import lax
from jax.experimental import pallas as pl
from jax.experimental.pallas import tpu as pltpu

_EPS = 1e-5


def _cparams(vmem_mb=64):
    return pltpu.CompilerParams(
        dimension_semantics=("parallel",),
        vmem_limit_bytes=vmem_mb * 2**20,
    )


def _stats2(y):
    return jnp.concatenate(
        [jnp.sum(y, axis=0, keepdims=True),
         jnp.sum(y * y, axis=0, keepdims=True)], axis=0)


def _bn_affine(st_sum, st_sq, g, b, count):
    mu = st_sum / count
    var = st_sq / count - mu * mu
    scale = g * lax.rsqrt(var + _EPS)
    shift = b - mu * scale
    return scale.reshape(1, -1), shift.reshape(1, -1)


def _stage_a_kernel(x_ref, w1b_ref, w0_ref, w2_ref, y1_ref, st_ref, xpad):
    l, cin = x_ref.shape[1], x_ref.shape[2]
    x = x_ref[0]
    xpad[0:1, :] = jnp.zeros((1, cin), jnp.float32)
    xpad[l + 1:l + 2, :] = jnp.zeros((1, cin), jnp.float32)
    xpad[1:l + 1, :] = x.astype(jnp.float32)
    p = jnp.dot(x, w1b_ref[...], preferred_element_type=jnp.float32)
    cout = w0_ref.shape[1]
    y1 = p[:, :cout]
    sc = p[:, cout:]
    y1 = y1 + jnp.dot(xpad[0:l, :].astype(jnp.bfloat16), w0_ref[...],
                      preferred_element_type=jnp.float32)
    y1 = y1 + jnp.dot(xpad[2:l + 2, :].astype(jnp.bfloat16), w2_ref[...],
                      preferred_element_type=jnp.float32)
    y1_ref[0] = y1.astype(jnp.bfloat16)
    st_ref[0, 0:2, :] = _stats2(y1)
    st_ref[0, 2:4, :] = _stats2(sc)
    st_ref[0, 4:8, :] = jnp.zeros((4, cout), jnp.float32)


_TAB = (((0,), (1,)), ((), ()))


def _stage_b_kernel(y1_ref, s1_ref, h1_ref, w0_ref, w1_ref, w2_ref,
                    z_ref, st_ref, apad):
    l, c = y1_ref.shape[1], y1_ref.shape[2]
    a = jnp.maximum(
        y1_ref[0].astype(jnp.float32) * s1_ref[...] + h1_ref[...], 0.0)
    apad[0:1, :] = jnp.zeros((1, c), jnp.float32)
    apad[l + 1:l + 2, :] = jnp.zeros((1, c), jnp.float32)
    apad[1:l + 1, :] = a
    z = lax.dot_general(w1_ref[...], a.astype(jnp.bfloat16), _TAB,
                        preferred_element_type=jnp.float32)
    z = z + lax.dot_general(w0_ref[...], apad[0:l, :].astype(jnp.bfloat16),
                            _TAB, preferred_element_type=jnp.float32)
    z = z + lax.dot_general(w2_ref[...], apad[2:l + 2, :].astype(jnp.bfloat16),
                            _TAB, preferred_element_type=jnp.float32)
    z_ref[0] = z.astype(jnp.bfloat16)
    st_ref[0, :, 0:1] = jnp.sum(z, axis=1, keepdims=True)
    st_ref[0, :, 1:2] = jnp.sum(z * z, axis=1, keepdims=True)
    st_ref[0, :, 2:8] = jnp.zeros((z.shape[0], 6), jnp.float32)


def _stage_c_kernel(z_ref, x_ref, ws_ref, s2_ref, h2_ref, ss_ref, hs_ref,
                    out_ref):
    sc = lax.dot_general(ws_ref[...], x_ref[0], _TAB,
                         preferred_element_type=jnp.float32)
    z = z_ref[0].astype(jnp.float32) * s2_ref[...] + h2_ref[...]
    s = sc * ss_ref[...] + hs_ref[...]
    out_ref[0] = jnp.maximum(z + s, 0.0)


@jax.jit
def _block(x, w1, w2, g1, b1, g2, b2, ws, gs, bs):
    N, Cin, L = x.shape
    Cout = w1.shape[0]
    count = float(N * L)

    x_cl = jnp.transpose(x, (0, 2, 1)).astype(jnp.bfloat16)
    w1t = jnp.transpose(w1, (2, 1, 0)).astype(jnp.bfloat16)
    w2t = jnp.transpose(w2, (2, 1, 0)).astype(jnp.bfloat16)
    wst = jnp.transpose(ws, (2, 1, 0))[0].astype(jnp.bfloat16)
    w1b = jnp.concatenate([w1t[1], wst], axis=1)

    def full(shp):
        n = len(shp)
        return pl.BlockSpec(shp, lambda b: (0,) * n)

    x_spec = pl.BlockSpec((1, L, Cin), lambda b: (b, 0, 0))
    row_spec = pl.BlockSpec((1, L, Cout), lambda b: (b, 0, 0))
    st_spec = pl.BlockSpec((1, 8, Cout), lambda b: (b, 0, 0))
    vec_spec = pl.BlockSpec((1, Cout), lambda b: (0, 0))

    row_bf = jax.ShapeDtypeStruct((N, L, Cout), jnp.bfloat16)
    st_struct = jax.ShapeDtypeStruct((N, 8, Cout), jnp.float32)

    y1, sta = pl.pallas_call(
        _stage_a_kernel,
        grid=(N,),
        in_specs=[x_spec, full((Cin, 2 * Cout)), full((Cin, Cout)),
                  full((Cin, Cout))],
        out_specs=[row_spec, st_spec],
        out_shape=[row_bf, st_struct],
        scratch_shapes=[pltpu.VMEM((L + 2, Cin), jnp.float32)],
        compiler_params=_cparams(),
    )(x_cl, w1b, w1t[0], w1t[2])

    st = jnp.sum(sta, axis=0)
    scale1, shift1 = _bn_affine(st[0], st[1], g1, b1, count)
    scale_s, shift_s = _bn_affine(st[2], st[3], gs, bs, count)

    cf_spec = pl.BlockSpec((1, Cout, L), lambda b: (b, 0, 0))
    stc_spec = pl.BlockSpec((1, Cout, 8), lambda b: (b, 0, 0))
    col_spec = pl.BlockSpec((Cout, 1), lambda b: (0, 0))

    z, stb = pl.pallas_call(
        _stage_b_kernel,
        grid=(N,),
        in_specs=[row_spec, vec_spec, vec_spec, full((Cout, Cout)),
                  full((Cout, Cout)), full((Cout, Cout))],
        out_specs=[cf_spec, stc_spec],
        out_shape=[jax.ShapeDtypeStruct((N, Cout, L), jnp.bfloat16),
                   jax.ShapeDtypeStruct((N, Cout, 8), jnp.float32)],
        scratch_shapes=[pltpu.VMEM((L + 2, Cout), jnp.float32)],
        compiler_params=_cparams(),
    )(y1, scale1, shift1, w2t[0], w2t[1], w2t[2])

    st2 = jnp.sum(stb, axis=0)
    scale2, shift2 = _bn_affine(st2[:, 0], st2[:, 1], g2, b2, count)

    out = pl.pallas_call(
        _stage_c_kernel,
        grid=(N,),
        in_specs=[cf_spec, x_spec, full((Cin, Cout)), col_spec, col_spec,
                  col_spec, col_spec],
        out_specs=cf_spec,
        out_shape=jax.ShapeDtypeStruct((N, Cout, L), jnp.float32),
        compiler_params=_cparams(),
    )(z, x_cl, wst, scale2.reshape(Cout, 1), shift2.reshape(Cout, 1),
      scale_s.reshape(Cout, 1), shift_s.reshape(Cout, 1))

    return out


def kernel(x, w1, w2, g1, b1, g2, b2, ws, gs, bs):
    return _block(x, w1, w2, g1, b1, g2, b2, ws, gs, bs)


# speedup vs baseline: 1.7143x; 1.2711x over previous
import functools
import math

import jax
import jax.numpy as jnp
from jax import lax
from jax.experimental import pallas as pl
from jax.experimental.pallas import tpu as pltpu

_EPS = 1e-5
_NB = 2


def _cparams(vmem_mb=96):
    return pltpu.CompilerParams(
        dimension_semantics=("parallel",),
        vmem_limit_bytes=vmem_mb * 2**20,
    )


def _stats2(y):
    return jnp.concatenate(
        [jnp.sum(y, axis=0, keepdims=True),
         jnp.sum(y * y, axis=0, keepdims=True)], axis=0)


def _bn_affine(st_sum, st_sq, g, b, count):
    mu = st_sum / count
    var = st_sq / count - mu * mu
    scale = g * lax.rsqrt(var + _EPS)
    shift = b - mu * scale
    return scale.reshape(1, -1), shift.reshape(1, -1)


_TAB = (((0,), (1,)), ((), ()))


def _stage_a_kernel(x_ref, w1b_ref, w0_ref, w2_ref, y1_ref, st_ref, xpad):
    l, cin = x_ref.shape[1], x_ref.shape[2]
    cout = w0_ref.shape[1]
    st = jnp.zeros((4, cout), jnp.float32)
    for i in range(_NB):
        x = x_ref[i]
        xpad[0:1, :] = jnp.zeros((1, cin), jnp.float32)
        xpad[l + 1:l + 2, :] = jnp.zeros((1, cin), jnp.float32)
        xpad[1:l + 1, :] = x.astype(jnp.float32)
        p = jnp.dot(x, w1b_ref[...], preferred_element_type=jnp.float32)
        y1 = p[:, :cout]
        sc = p[:, cout:]
        y1 = y1 + jnp.dot(xpad[0:l, :].astype(jnp.bfloat16), w0_ref[...],
                          preferred_element_type=jnp.float32)
        y1 = y1 + jnp.dot(xpad[2:l + 2, :].astype(jnp.bfloat16), w2_ref[...],
                          preferred_element_type=jnp.float32)
        y1_ref[i] = y1.astype(jnp.bfloat16)
        st = st + jnp.concatenate([_stats2(y1), _stats2(sc)], axis=0)
    st_ref[0, 0:4, :] = st
    st_ref[0, 4:8, :] = jnp.zeros((4, cout), jnp.float32)


def _stage_b_kernel(y1_ref, s1_ref, h1_ref, w0_ref, w1_ref, w2_ref,
                    z_ref, st_ref, apad):
    l, c = y1_ref.shape[1], y1_ref.shape[2]
    st_sum = jnp.zeros((c, 1), jnp.float32)
    st_sq = jnp.zeros((c, 1), jnp.float32)
    for i in range(_NB):
        a = jnp.maximum(
            y1_ref[i].astype(jnp.float32) * s1_ref[...] + h1_ref[...], 0.0)
        apad[0:1, :] = jnp.zeros((1, c), jnp.float32)
        apad[l + 1:l + 2, :] = jnp.zeros((1, c), jnp.float32)
        apad[1:l + 1, :] = a
        z = lax.dot_general(w1_ref[...], a.astype(jnp.bfloat16), _TAB,
                            preferred_element_type=jnp.float32)
        z = z + lax.dot_general(w0_ref[...],
                                apad[0:l, :].astype(jnp.bfloat16),
                                _TAB, preferred_element_type=jnp.float32)
        z = z + lax.dot_general(w2_ref[...],
                                apad[2:l + 2, :].astype(jnp.bfloat16),
                                _TAB, preferred_element_type=jnp.float32)
        z_ref[i] = z.astype(jnp.bfloat16)
        st_sum = st_sum + jnp.sum(z, axis=1, keepdims=True)
        st_sq = st_sq + jnp.sum(z * z, axis=1, keepdims=True)
    st_ref[0, :, 0:1] = st_sum
    st_ref[0, :, 1:2] = st_sq
    st_ref[0, :, 2:8] = jnp.zeros((c, 6), jnp.float32)


def _stage_c_kernel(z_ref, x_ref, ws_ref, s2_ref, h2_ref, ss_ref, hs_ref,
                    out_ref):
    for i in range(_NB):
        sc = lax.dot_general(ws_ref[...], x_ref[i], _TAB,
                             preferred_element_type=jnp.float32)
        z = z_ref[i].astype(jnp.float32) * s2_ref[...] + h2_ref[...]
        s = sc * ss_ref[...] + hs_ref[...]
        out_ref[i] = jnp.maximum(z + s, 0.0)


@jax.jit
def _block(x, w1, w2, g1, b1, g2, b2, ws, gs, bs):
    N, Cin, L = x.shape
    Cout = w1.shape[0]
    count = float(N * L)
    G = N // _NB

    x_cl = jnp.transpose(x, (0, 2, 1)).astype(jnp.bfloat16)
    w1t = jnp.transpose(w1, (2, 1, 0)).astype(jnp.bfloat16)
    w2t = jnp.transpose(w2, (2, 1, 0)).astype(jnp.bfloat16)
    wst = jnp.transpose(ws, (2, 1, 0))[0].astype(jnp.bfloat16)
    w1b = jnp.concatenate([w1t[1], wst], axis=1)

    def full(shp):
        n = len(shp)
        return pl.BlockSpec(shp, lambda b: (0,) * n)

    x_spec = pl.BlockSpec((_NB, L, Cin), lambda b: (b, 0, 0))
    row_spec = pl.BlockSpec((_NB, L, Cout), lambda b: (b, 0, 0))
    st_spec = pl.BlockSpec((1, 8, Cout), lambda b: (b, 0, 0))
    vec_spec = pl.BlockSpec((1, Cout), lambda b: (0, 0))

    y1, sta = pl.pallas_call(
        _stage_a_kernel,
        grid=(G,),
        in_specs=[x_spec, full((Cin, 2 * Cout)), full((Cin, Cout)),
                  full((Cin, Cout))],
        out_specs=[row_spec, st_spec],
        out_shape=[jax.ShapeDtypeStruct((N, L, Cout), jnp.bfloat16),
                   jax.ShapeDtypeStruct((G, 8, Cout), jnp.float32)],
        scratch_shapes=[pltpu.VMEM((L + 2, Cin), jnp.float32)],
        compiler_params=_cparams(),
    )(x_cl, w1b, w1t[0], w1t[2])

    st = jnp.sum(sta, axis=0)
    scale1, shift1 = _bn_affine(st[0], st[1], g1, b1, count)
    scale_s, shift_s = _bn_affine(st[2], st[3], gs, bs, count)

    cf_spec = pl.BlockSpec((_NB, Cout, L), lambda b: (b, 0, 0))
    stc_spec = pl.BlockSpec((1, Cout, 8), lambda b: (b, 0, 0))
    col_spec = pl.BlockSpec((Cout, 1), lambda b: (0, 0))

    z, stb = pl.pallas_call(
        _stage_b_kernel,
        grid=(G,),
        in_specs=[row_spec, vec_spec, vec_spec, full((Cout, Cout)),
                  full((Cout, Cout)), full((Cout, Cout))],
        out_specs=[cf_spec, stc_spec],
        out_shape=[jax.ShapeDtypeStruct((N, Cout, L), jnp.bfloat16),
                   jax.ShapeDtypeStruct((G, Cout, 8), jnp.float32)],
        scratch_shapes=[pltpu.VMEM((L + 2, Cout), jnp.float32)],
        compiler_params=_cparams(),
    )(y1, scale1, shift1, w2t[0], w2t[1], w2t[2])

    st2 = jnp.sum(stb, axis=0)
    scale2, shift2 = _bn_affine(st2[:, 0], st2[:, 1], g2, b2, count)

    out = pl.pallas_call(
        _stage_c_kernel,
        grid=(G,),
        in_specs=[cf_spec, x_spec, full((Cin, Cout)), col_spec, col_spec,
                  col_spec, col_spec],
        out_specs=cf_spec,
        out_shape=jax.ShapeDtypeStruct((N, Cout, L), jnp.float32),
        compiler_params=_cparams(),
    )(z, x_cl, wst, scale2.reshape(Cout, 1), shift2.reshape(Cout, 1),
      scale_s.reshape(Cout, 1), shift_s.reshape(Cout, 1))

    return out


def kernel(x, w1, w2, g1, b1, g2, b2, ws, gs, bs):
    return _block(x, w1, w2, g1, b1, g2, b2, ws, gs, bs)


# speedup vs baseline: 1.9210x; 1.1206x over previous
import functools
import math

import jax
import jax.numpy as jnp
from jax import lax
from jax.experimental import pallas as pl
from jax.experimental.pallas import tpu as pltpu

_EPS = 1e-5
_NB = 4


def _cparams(vmem_mb=96):
    return pltpu.CompilerParams(
        dimension_semantics=("parallel",),
        vmem_limit_bytes=vmem_mb * 2**20,
    )


def _stats2(y):
    return jnp.concatenate(
        [jnp.sum(y, axis=0, keepdims=True),
         jnp.sum(y * y, axis=0, keepdims=True)], axis=0)


def _bn_affine(st_sum, st_sq, g, b, count):
    mu = st_sum / count
    var = st_sq / count - mu * mu
    scale = g * lax.rsqrt(var + _EPS)
    shift = b - mu * scale
    return scale.reshape(1, -1), shift.reshape(1, -1)


_TAB = (((0,), (1,)), ((), ()))


def _stage_a_kernel(x_ref, w1b_ref, w0_ref, w2_ref, y1_ref, st_ref, xpad):
    l, cin = x_ref.shape[1], x_ref.shape[2]
    cout = w0_ref.shape[1]
    st = jnp.zeros((4, cout), jnp.float32)
    for i in range(_NB):
        x = x_ref[i]
        xpad[0:1, :] = jnp.zeros((1, cin), jnp.float32)
        xpad[l + 1:l + 2, :] = jnp.zeros((1, cin), jnp.float32)
        xpad[1:l + 1, :] = x.astype(jnp.float32)
        p = jnp.dot(x, w1b_ref[...], preferred_element_type=jnp.float32)
        y1 = p[:, :cout]
        sc = p[:, cout:]
        y1 = y1 + jnp.dot(xpad[0:l, :].astype(jnp.bfloat16), w0_ref[...],
                          preferred_element_type=jnp.float32)
        y1 = y1 + jnp.dot(xpad[2:l + 2, :].astype(jnp.bfloat16), w2_ref[...],
                          preferred_element_type=jnp.float32)
        y1_ref[i] = y1.astype(jnp.bfloat16)
        st = st + jnp.concatenate([_stats2(y1), _stats2(sc)], axis=0)
    st_ref[0, 0:4, :] = st
    st_ref[0, 4:8, :] = jnp.zeros((4, cout), jnp.float32)


def _stage_b_kernel(y1_ref, s1_ref, h1_ref, w0_ref, w1_ref, w2_ref,
                    z_ref, st_ref, apad):
    l, c = y1_ref.shape[1], y1_ref.shape[2]
    st_sum = jnp.zeros((c, 1), jnp.float32)
    st_sq = jnp.zeros((c, 1), jnp.float32)
    for i in range(_NB):
        a = jnp.maximum(
            y1_ref[i].astype(jnp.float32) * s1_ref[...] + h1_ref[...], 0.0)
        apad[0:1, :] = jnp.zeros((1, c), jnp.float32)
        apad[l + 1:l + 2, :] = jnp.zeros((1, c), jnp.float32)
        apad[1:l + 1, :] = a
        z = lax.dot_general(w1_ref[...], a.astype(jnp.bfloat16), _TAB,
                            preferred_element_type=jnp.float32)
        z = z + lax.dot_general(w0_ref[...],
                                apad[0:l, :].astype(jnp.bfloat16),
                                _TAB, preferred_element_type=jnp.float32)
        z = z + lax.dot_general(w2_ref[...],
                                apad[2:l + 2, :].astype(jnp.bfloat16),
                                _TAB, preferred_element_type=jnp.float32)
        z_ref[i] = z.astype(jnp.bfloat16)
        st_sum = st_sum + jnp.sum(z, axis=1, keepdims=True)
        st_sq = st_sq + jnp.sum(z * z, axis=1, keepdims=True)
    st_ref[0, :, 0:1] = st_sum
    st_ref[0, :, 1:2] = st_sq
    st_ref[0, :, 2:8] = jnp.zeros((c, 6), jnp.float32)


def _stage_c_kernel(z_ref, x_ref, ws_ref, s2_ref, h2_ref, ss_ref, hs_ref,
                    out_ref):
    for i in range(_NB):
        sc = lax.dot_general(ws_ref[...], x_ref[i], _TAB,
                             preferred_element_type=jnp.float32)
        z = z_ref[i].astype(jnp.float32) * s2_ref[...] + h2_ref[...]
        s = sc * ss_ref[...] + hs_ref[...]
        out_ref[i] = jnp.maximum(z + s, 0.0)


@jax.jit
def _block(x, w1, w2, g1, b1, g2, b2, ws, gs, bs):
    N, Cin, L = x.shape
    Cout = w1.shape[0]
    count = float(N * L)
    G = N // _NB

    x_cl = jnp.transpose(x, (0, 2, 1)).astype(jnp.bfloat16)
    w1t = jnp.transpose(w1, (2, 1, 0)).astype(jnp.bfloat16)
    w2t = jnp.transpose(w2, (2, 1, 0)).astype(jnp.bfloat16)
    wst = jnp.transpose(ws, (2, 1, 0))[0].astype(jnp.bfloat16)
    w1b = jnp.concatenate([w1t[1], wst], axis=1)

    def full(shp):
        n = len(shp)
        return pl.BlockSpec(shp, lambda b: (0,) * n)

    x_spec = pl.BlockSpec((_NB, L, Cin), lambda b: (b, 0, 0))
    row_spec = pl.BlockSpec((_NB, L, Cout), lambda b: (b, 0, 0))
    st_spec = pl.BlockSpec((1, 8, Cout), lambda b: (b, 0, 0))
    vec_spec = pl.BlockSpec((1, Cout), lambda b: (0, 0))

    y1, sta = pl.pallas_call(
        _stage_a_kernel,
        grid=(G,),
        in_specs=[x_spec, full((Cin, 2 * Cout)), full((Cin, Cout)),
                  full((Cin, Cout))],
        out_specs=[row_spec, st_spec],
        out_shape=[jax.ShapeDtypeStruct((N, L, Cout), jnp.bfloat16),
                   jax.ShapeDtypeStruct((G, 8, Cout), jnp.float32)],
        scratch_shapes=[pltpu.VMEM((L + 2, Cin), jnp.float32)],
        compiler_params=_cparams(),
    )(x_cl, w1b, w1t[0], w1t[2])

    st = jnp.sum(sta, axis=0)
    scale1, shift1 = _bn_affine(st[0], st[1], g1, b1, count)
    scale_s, shift_s = _bn_affine(st[2], st[3], gs, bs, count)

    cf_spec = pl.BlockSpec((_NB, Cout, L), lambda b: (b, 0, 0))
    stc_spec = pl.BlockSpec((1, Cout, 8), lambda b: (b, 0, 0))
    col_spec = pl.BlockSpec((Cout, 1), lambda b: (0, 0))

    z, stb = pl.pallas_call(
        _stage_b_kernel,
        grid=(G,),
        in_specs=[row_spec, vec_spec, vec_spec, full((Cout, Cout)),
                  full((Cout, Cout)), full((Cout, Cout))],
        out_specs=[cf_spec, stc_spec],
        out_shape=[jax.ShapeDtypeStruct((N, Cout, L), jnp.bfloat16),
                   jax.ShapeDtypeStruct((G, Cout, 8), jnp.float32)],
        scratch_shapes=[pltpu.VMEM((L + 2, Cout), jnp.float32)],
        compiler_params=_cparams(),
    )(y1, scale1, shift1, w2t[0], w2t[1], w2t[2])

    st2 = jnp.sum(stb, axis=0)
    scale2, shift2 = _bn_affine(st2[:, 0], st2[:, 1], g2, b2, count)

    out = pl.pallas_call(
        _stage_c_kernel,
        grid=(G,),
        in_specs=[cf_spec, x_spec, full((Cin, Cout)), col_spec, col_spec,
                  col_spec, col_spec],
        out_specs=cf_spec,
        out_shape=jax.ShapeDtypeStruct((N, Cout, L), jnp.float32),
        compiler_params=_cparams(),
    )(z, x_cl, wst, scale2.reshape(Cout, 1), shift2.reshape(Cout, 1),
      scale_s.reshape(Cout, 1), shift_s.reshape(Cout, 1))

    return out


def kernel(x, w1, w2, g1, b1, g2, b2, ws, gs, bs):
    return _block(x, w1, w2, g1, b1, g2, b2, ws, gs, bs)


# speedup vs baseline: 1.9636x; 1.0222x over previous
import functools
import math

import jax
import jax.numpy as jnp
from jax import lax
from jax.experimental import pallas as pl
from jax.experimental.pallas import tpu as pltpu

_EPS = 1e-5
_NB = 8


def _cparams(vmem_mb=96):
    return pltpu.CompilerParams(
        dimension_semantics=("parallel",),
        vmem_limit_bytes=vmem_mb * 2**20,
    )


def _stats2(y):
    return jnp.concatenate(
        [jnp.sum(y, axis=0, keepdims=True),
         jnp.sum(y * y, axis=0, keepdims=True)], axis=0)


def _bn_affine(st_sum, st_sq, g, b, count):
    mu = st_sum / count
    var = st_sq / count - mu * mu
    scale = g * lax.rsqrt(var + _EPS)
    shift = b - mu * scale
    return scale.reshape(1, -1), shift.reshape(1, -1)


_TAB = (((0,), (1,)), ((), ()))


def _stage_a_kernel(x_ref, w1b_ref, w0_ref, w2_ref, y1_ref, st_ref, xpad):
    l, cin = x_ref.shape[1], x_ref.shape[2]
    cout = w0_ref.shape[1]
    st = jnp.zeros((4, cout), jnp.float32)
    for i in range(_NB):
        x = x_ref[i]
        xpad[0:1, :] = jnp.zeros((1, cin), jnp.float32)
        xpad[l + 1:l + 2, :] = jnp.zeros((1, cin), jnp.float32)
        xpad[1:l + 1, :] = x.astype(jnp.float32)
        p = jnp.dot(x, w1b_ref[...], preferred_element_type=jnp.float32)
        y1 = p[:, :cout]
        sc = p[:, cout:]
        y1 = y1 + jnp.dot(xpad[0:l, :].astype(jnp.bfloat16), w0_ref[...],
                          preferred_element_type=jnp.float32)
        y1 = y1 + jnp.dot(xpad[2:l + 2, :].astype(jnp.bfloat16), w2_ref[...],
                          preferred_element_type=jnp.float32)
        y1_ref[i] = y1.astype(jnp.bfloat16)
        st = st + jnp.concatenate([_stats2(y1), _stats2(sc)], axis=0)
    st_ref[0, 0:4, :] = st
    st_ref[0, 4:8, :] = jnp.zeros((4, cout), jnp.float32)


def _stage_b_kernel(y1_ref, s1_ref, h1_ref, w0_ref, w1_ref, w2_ref,
                    z_ref, st_ref, apad):
    l, c = y1_ref.shape[1], y1_ref.shape[2]
    st_sum = jnp.zeros((c, 1), jnp.float32)
    st_sq = jnp.zeros((c, 1), jnp.float32)
    for i in range(_NB):
        a = jnp.maximum(
            y1_ref[i].astype(jnp.float32) * s1_ref[...] + h1_ref[...], 0.0)
        apad[0:1, :] = jnp.zeros((1, c), jnp.float32)
        apad[l + 1:l + 2, :] = jnp.zeros((1, c), jnp.float32)
        apad[1:l + 1, :] = a
        z = lax.dot_general(w1_ref[...], a.astype(jnp.bfloat16), _TAB,
                            preferred_element_type=jnp.float32)
        z = z + lax.dot_general(w0_ref[...],
                                apad[0:l, :].astype(jnp.bfloat16),
                                _TAB, preferred_element_type=jnp.float32)
        z = z + lax.dot_general(w2_ref[...],
                                apad[2:l + 2, :].astype(jnp.bfloat16),
                                _TAB, preferred_element_type=jnp.float32)
        z_ref[i] = z.astype(jnp.bfloat16)
        st_sum = st_sum + jnp.sum(z, axis=1, keepdims=True)
        st_sq = st_sq + jnp.sum(z * z, axis=1, keepdims=True)
    st_ref[0, :, 0:1] = st_sum
    st_ref[0, :, 1:2] = st_sq
    st_ref[0, :, 2:8] = jnp.zeros((c, 6), jnp.float32)


def _stage_c_kernel(z_ref, x_ref, ws_ref, s2_ref, h2_ref, ss_ref, hs_ref,
                    out_ref):
    for i in range(_NB):
        sc = lax.dot_general(ws_ref[...], x_ref[i], _TAB,
                             preferred_element_type=jnp.float32)
        z = z_ref[i].astype(jnp.float32) * s2_ref[...] + h2_ref[...]
        s = sc * ss_ref[...] + hs_ref[...]
        out_ref[i] = jnp.maximum(z + s, 0.0)


@jax.jit
def _block(x, w1, w2, g1, b1, g2, b2, ws, gs, bs):
    N, Cin, L = x.shape
    Cout = w1.shape[0]
    count = float(N * L)
    G = N // _NB

    x_cl = jnp.transpose(x, (0, 2, 1)).astype(jnp.bfloat16)
    w1t = jnp.transpose(w1, (2, 1, 0)).astype(jnp.bfloat16)
    w2t = jnp.transpose(w2, (2, 1, 0)).astype(jnp.bfloat16)
    wst = jnp.transpose(ws, (2, 1, 0))[0].astype(jnp.bfloat16)
    w1b = jnp.concatenate([w1t[1], wst], axis=1)

    def full(shp):
        n = len(shp)
        return pl.BlockSpec(shp, lambda b: (0,) * n)

    x_spec = pl.BlockSpec((_NB, L, Cin), lambda b: (b, 0, 0))
    row_spec = pl.BlockSpec((_NB, L, Cout), lambda b: (b, 0, 0))
    st_spec = pl.BlockSpec((1, 8, Cout), lambda b: (b, 0, 0))
    vec_spec = pl.BlockSpec((1, Cout), lambda b: (0, 0))

    y1, sta = pl.pallas_call(
        _stage_a_kernel,
        grid=(G,),
        in_specs=[x_spec, full((Cin, 2 * Cout)), full((Cin, Cout)),
                  full((Cin, Cout))],
        out_specs=[row_spec, st_spec],
        out_shape=[jax.ShapeDtypeStruct((N, L, Cout), jnp.bfloat16),
                   jax.ShapeDtypeStruct((G, 8, Cout), jnp.float32)],
        scratch_shapes=[pltpu.VMEM((L + 2, Cin), jnp.float32)],
        compiler_params=_cparams(),
    )(x_cl, w1b, w1t[0], w1t[2])

    st = jnp.sum(sta, axis=0)
    scale1, shift1 = _bn_affine(st[0], st[1], g1, b1, count)
    scale_s, shift_s = _bn_affine(st[2], st[3], gs, bs, count)

    cf_spec = pl.BlockSpec((_NB, Cout, L), lambda b: (b, 0, 0))
    stc_spec = pl.BlockSpec((1, Cout, 8), lambda b: (b, 0, 0))
    col_spec = pl.BlockSpec((Cout, 1), lambda b: (0, 0))

    z, stb = pl.pallas_call(
        _stage_b_kernel,
        grid=(G,),
        in_specs=[row_spec, vec_spec, vec_spec, full((Cout, Cout)),
                  full((Cout, Cout)), full((Cout, Cout))],
        out_specs=[cf_spec, stc_spec],
        out_shape=[jax.ShapeDtypeStruct((N, Cout, L), jnp.bfloat16),
                   jax.ShapeDtypeStruct((G, Cout, 8), jnp.float32)],
        scratch_shapes=[pltpu.VMEM((L + 2, Cout), jnp.float32)],
        compiler_params=_cparams(),
    )(y1, scale1, shift1, w2t[0], w2t[1], w2t[2])

    st2 = jnp.sum(stb, axis=0)
    scale2, shift2 = _bn_affine(st2[:, 0], st2[:, 1], g2, b2, count)

    out = pl.pallas_call(
        _stage_c_kernel,
        grid=(G,),
        in_specs=[cf_spec, x_spec, full((Cin, Cout)), col_spec, col_spec,
                  col_spec, col_spec],
        out_specs=cf_spec,
        out_shape=jax.ShapeDtypeStruct((N, Cout, L), jnp.float32),
        compiler_params=_cparams(),
    )(z, x_cl, wst, scale2.reshape(Cout, 1), shift2.reshape(Cout, 1),
      scale_s.reshape(Cout, 1), shift_s.reshape(Cout, 1))

    return out


def kernel(x, w1, w2, g1, b1, g2, b2, ws, gs, bs):
    return _block(x, w1, w2, g1, b1, g2, b2, ws, gs, bs)


# speedup vs baseline: 2.1857x; 1.1131x over previous
import functools
import math

import jax
import jax.numpy as jnp
from jax import lax
from jax.experimental import pallas as pl
from jax.experimental.pallas import tpu as pltpu

_EPS = 1e-5
_NB = 8


def _cparams(vmem_mb=96):
    return pltpu.CompilerParams(
        dimension_semantics=("parallel",),
        vmem_limit_bytes=vmem_mb * 2**20,
    )


def _stats2(y):
    return jnp.concatenate(
        [jnp.sum(y, axis=0, keepdims=True),
         jnp.sum(y * y, axis=0, keepdims=True)], axis=0)


def _bn_affine(st_sum, st_sq, g, b, count):
    mu = st_sum / count
    var = st_sq / count - mu * mu
    scale = g * lax.rsqrt(var + _EPS)
    shift = b - mu * scale
    return scale.reshape(1, -1), shift.reshape(1, -1)


_TAB = (((0,), (1,)), ((), ()))


_TA = (((0,), (0,)), ((), ()))


def _stage_a_kernel(x_ref, w1b_ref, w0_ref, w2_ref, xbf_ref, y1_ref, st_ref):
    cin = x_ref.shape[1]
    cout = w0_ref.shape[1]
    st = jnp.zeros((4, cout), jnp.float32)
    zcol = jnp.zeros((cin, 1), jnp.bfloat16)
    for i in range(x_ref.shape[0]):
        x = x_ref[i].astype(jnp.bfloat16)
        xbf_ref[i] = x
        s0 = jnp.concatenate([zcol, x[:, :-1]], axis=1)
        s2 = jnp.concatenate([x[:, 1:], zcol], axis=1)
        p = lax.dot_general(x, w1b_ref[...], _TA,
                            preferred_element_type=jnp.float32)
        y1 = p[:, :cout]
        sc = p[:, cout:]
        y1 = y1 + lax.dot_general(s0, w0_ref[...], _TA,
                                  preferred_element_type=jnp.float32)
        y1 = y1 + lax.dot_general(s2, w2_ref[...], _TA,
                                  preferred_element_type=jnp.float32)
        y1_ref[i] = y1.astype(jnp.bfloat16)
        st = st + jnp.concatenate([_stats2(y1), _stats2(sc)], axis=0)
    st_ref[0, 0:4, :] = st
    st_ref[0, 4:8, :] = jnp.zeros((4, cout), jnp.float32)


def _stage_b_kernel(y1_ref, s1_ref, h1_ref, w0_ref, w1_ref, w2_ref,
                    z_ref, st_ref, apad):
    l, c = y1_ref.shape[1], y1_ref.shape[2]
    st_sum = jnp.zeros((c, 1), jnp.float32)
    st_sq = jnp.zeros((c, 1), jnp.float32)
    for i in range(y1_ref.shape[0]):
        a = jnp.maximum(
            y1_ref[i].astype(jnp.float32) * s1_ref[...] + h1_ref[...], 0.0)
        apad[0:1, :] = jnp.zeros((1, c), jnp.float32)
        apad[l + 1:l + 2, :] = jnp.zeros((1, c), jnp.float32)
        apad[1:l + 1, :] = a
        z = lax.dot_general(w1_ref[...], a.astype(jnp.bfloat16), _TAB,
                            preferred_element_type=jnp.float32)
        z = z + lax.dot_general(w0_ref[...],
                                apad[0:l, :].astype(jnp.bfloat16),
                                _TAB, preferred_element_type=jnp.float32)
        z = z + lax.dot_general(w2_ref[...],
                                apad[2:l + 2, :].astype(jnp.bfloat16),
                                _TAB, preferred_element_type=jnp.float32)
        z_ref[i] = z.astype(jnp.bfloat16)
        st_sum = st_sum + jnp.sum(z, axis=1, keepdims=True)
        st_sq = st_sq + jnp.sum(z * z, axis=1, keepdims=True)
    st_ref[0, :, 0:1] = st_sum
    st_ref[0, :, 1:2] = st_sq
    st_ref[0, :, 2:8] = jnp.zeros((c, 6), jnp.float32)


def _stage_c_kernel(z_ref, x_ref, ws_ref, s2_ref, h2_ref, ss_ref, hs_ref,
                    out_ref):
    for i in range(z_ref.shape[0]):
        sc = lax.dot_general(ws_ref[...], x_ref[i],
                             (((0,), (0,)), ((), ())),
                             preferred_element_type=jnp.float32)
        z = z_ref[i].astype(jnp.float32) * s2_ref[...] + h2_ref[...]
        s = sc * ss_ref[...] + hs_ref[...]
        out_ref[i] = jnp.maximum(z + s, 0.0)


@jax.jit
def _block(x, w1, w2, g1, b1, g2, b2, ws, gs, bs):
    N, Cin, L = x.shape
    Cout = w1.shape[0]
    count = float(N * L)
    nb = min(_NB, N)
    G = N // nb

    w1t = jnp.transpose(w1, (2, 1, 0)).astype(jnp.bfloat16)
    w2t = jnp.transpose(w2, (2, 1, 0)).astype(jnp.bfloat16)
    wst = jnp.transpose(ws, (2, 1, 0))[0].astype(jnp.bfloat16)
    w1b = jnp.concatenate([w1t[1], wst], axis=1)

    def full(shp):
        n = len(shp)
        return pl.BlockSpec(shp, lambda b: (0,) * n)

    xin_spec = pl.BlockSpec((nb, Cin, L), lambda b: (b, 0, 0))
    row_spec = pl.BlockSpec((nb, L, Cout), lambda b: (b, 0, 0))
    st_spec = pl.BlockSpec((1, 8, Cout), lambda b: (b, 0, 0))
    vec_spec = pl.BlockSpec((1, Cout), lambda b: (0, 0))

    xbf, y1, sta = pl.pallas_call(
        _stage_a_kernel,
        grid=(G,),
        in_specs=[xin_spec, full((Cin, 2 * Cout)), full((Cin, Cout)),
                  full((Cin, Cout))],
        out_specs=[xin_spec, row_spec, st_spec],
        out_shape=[jax.ShapeDtypeStruct((N, Cin, L), jnp.bfloat16),
                   jax.ShapeDtypeStruct((N, L, Cout), jnp.bfloat16),
                   jax.ShapeDtypeStruct((G, 8, Cout), jnp.float32)],
        compiler_params=_cparams(),
    )(x, w1b, w1t[0], w1t[2])

    st = jnp.sum(sta, axis=0)
    scale1, shift1 = _bn_affine(st[0], st[1], g1, b1, count)
    scale_s, shift_s = _bn_affine(st[2], st[3], gs, bs, count)

    cf_spec = pl.BlockSpec((nb, Cout, L), lambda b: (b, 0, 0))
    stc_spec = pl.BlockSpec((1, Cout, 8), lambda b: (b, 0, 0))
    col_spec = pl.BlockSpec((Cout, 1), lambda b: (0, 0))

    z, stb = pl.pallas_call(
        _stage_b_kernel,
        grid=(G,),
        in_specs=[row_spec, vec_spec, vec_spec, full((Cout, Cout)),
                  full((Cout, Cout)), full((Cout, Cout))],
        out_specs=[cf_spec, stc_spec],
        out_shape=[jax.ShapeDtypeStruct((N, Cout, L), jnp.bfloat16),
                   jax.ShapeDtypeStruct((G, Cout, 8), jnp.float32)],
        scratch_shapes=[pltpu.VMEM((L + 2, Cout), jnp.float32)],
        compiler_params=_cparams(),
    )(y1, scale1, shift1, w2t[0], w2t[1], w2t[2])

    st2 = jnp.sum(stb, axis=0)
    scale2, shift2 = _bn_affine(st2[:, 0], st2[:, 1], g2, b2, count)

    out = pl.pallas_call(
        _stage_c_kernel,
        grid=(G,),
        in_specs=[cf_spec, xin_spec, full((Cin, Cout)), col_spec, col_spec,
                  col_spec, col_spec],
        out_specs=cf_spec,
        out_shape=jax.ShapeDtypeStruct((N, Cout, L), jnp.float32),
        compiler_params=_cparams(),
    )(z, xbf, wst, scale2.reshape(Cout, 1), shift2.reshape(Cout, 1),
      scale_s.reshape(Cout, 1), shift_s.reshape(Cout, 1))

    return out


def kernel(x, w1, w2, g1, b1, g2, b2, ws, gs, bs):
    return _block(x, w1, w2, g1, b1, g2, b2, ws, gs, bs)


# speedup vs baseline: 2.2309x; 1.0207x over previous
import functools
import math

import jax
import jax.numpy as jnp
from jax import lax
from jax.experimental import pallas as pl
from jax.experimental.pallas import tpu as pltpu

_EPS = 1e-5
_NB = 8


def _cparams(vmem_mb=96):
    return pltpu.CompilerParams(
        dimension_semantics=("parallel",),
        vmem_limit_bytes=vmem_mb * 2**20,
    )


def _stats2(y):
    return jnp.concatenate(
        [jnp.sum(y, axis=0, keepdims=True),
         jnp.sum(y * y, axis=0, keepdims=True)], axis=0)


def _bn_affine(st_sum, st_sq, g, b, count):
    mu = st_sum / count
    var = st_sq / count - mu * mu
    scale = g * lax.rsqrt(var + _EPS)
    shift = b - mu * scale
    return scale.reshape(1, -1), shift.reshape(1, -1)


_TAB = (((0,), (1,)), ((), ()))


_TA = (((0,), (0,)), ((), ()))


def _stage_a_kernel(x_ref, w1b_ref, w0_ref, w2_ref, xbf_ref, y1_ref, st_ref):
    cin = x_ref.shape[1]
    cout = w0_ref.shape[1]
    st = jnp.zeros((4, cout), jnp.float32)
    zcol = jnp.zeros((cin, 1), jnp.bfloat16)
    for i in range(x_ref.shape[0]):
        x = x_ref[i].astype(jnp.bfloat16)
        xbf_ref[i] = x
        s0 = jnp.concatenate([zcol, x[:, :-1]], axis=1)
        s2 = jnp.concatenate([x[:, 1:], zcol], axis=1)
        p = lax.dot_general(x, w1b_ref[...], _TA,
                            preferred_element_type=jnp.float32)
        y1 = p[:, :cout]
        sc = p[:, cout:]
        y1 = y1 + lax.dot_general(s0, w0_ref[...], _TA,
                                  preferred_element_type=jnp.float32)
        y1 = y1 + lax.dot_general(s2, w2_ref[...], _TA,
                                  preferred_element_type=jnp.float32)
        y1_ref[i] = y1.astype(jnp.bfloat16)
        st = st + jnp.concatenate([_stats2(y1), _stats2(sc)], axis=0)
    st_ref[0, 0:4, :] = st
    st_ref[0, 4:8, :] = jnp.zeros((4, cout), jnp.float32)


def _stage_b_kernel(y1_ref, sta_ref, g1_ref, b1_ref, w0_ref, w1_ref, w2_ref,
                    z_ref, st_ref, apad, *, count):
    l, c = y1_ref.shape[1], y1_ref.shape[2]
    stv = jnp.sum(sta_ref[...], axis=0)
    mu = stv[0:1] / count
    var = stv[1:2] / count - mu * mu
    s1 = g1_ref[...] * lax.rsqrt(var + _EPS)
    h1 = b1_ref[...] - mu * s1
    st_sum = jnp.zeros((c, 1), jnp.float32)
    st_sq = jnp.zeros((c, 1), jnp.float32)
    for i in range(y1_ref.shape[0]):
        a = jnp.maximum(y1_ref[i].astype(jnp.float32) * s1 + h1, 0.0)
        apad[0:1, :] = jnp.zeros((1, c), jnp.float32)
        apad[l + 1:l + 2, :] = jnp.zeros((1, c), jnp.float32)
        apad[1:l + 1, :] = a
        z = lax.dot_general(w1_ref[...], a.astype(jnp.bfloat16), _TAB,
                            preferred_element_type=jnp.float32)
        z = z + lax.dot_general(w0_ref[...],
                                apad[0:l, :].astype(jnp.bfloat16),
                                _TAB, preferred_element_type=jnp.float32)
        z = z + lax.dot_general(w2_ref[...],
                                apad[2:l + 2, :].astype(jnp.bfloat16),
                                _TAB, preferred_element_type=jnp.float32)
        z_ref[i] = z.astype(jnp.bfloat16)
        st_sum = st_sum + jnp.sum(z, axis=1, keepdims=True)
        st_sq = st_sq + jnp.sum(z * z, axis=1, keepdims=True)
    st_ref[0, :, 0:1] = st_sum
    st_ref[0, :, 1:2] = st_sq
    st_ref[0, :, 2:8] = jnp.zeros((c, 6), jnp.float32)


def _stage_c_kernel(z_ref, x_ref, ws_ref, stb_ref, sta_ref, g2_ref, b2_ref,
                    gs_ref, bs_ref, out_ref, *, count):
    stv = jnp.sum(stb_ref[...], axis=0)
    mu2 = stv[:, 0:1] / count
    var2 = stv[:, 1:2] / count - mu2 * mu2
    s2 = g2_ref[...] * lax.rsqrt(var2 + _EPS)
    h2 = b2_ref[...] - mu2 * s2
    sts = jnp.sum(sta_ref[...], axis=0)
    mus = jnp.transpose(sts[2:3] / count)
    vars_ = jnp.transpose(sts[3:4] / count) - mus * mus
    ss = gs_ref[...] * lax.rsqrt(vars_ + _EPS)
    hs = bs_ref[...] - mus * ss
    for i in range(z_ref.shape[0]):
        sc = lax.dot_general(ws_ref[...], x_ref[i],
                             (((0,), (0,)), ((), ())),
                             preferred_element_type=jnp.float32)
        z = z_ref[i].astype(jnp.float32) * s2 + h2
        s = sc * ss + hs
        out_ref[i] = jnp.maximum(z + s, 0.0)


@jax.jit
def _block(x, w1, w2, g1, b1, g2, b2, ws, gs, bs):
    N, Cin, L = x.shape
    Cout = w1.shape[0]
    count = float(N * L)
    nb = min(_NB, N)
    G = N // nb

    w1t = jnp.transpose(w1, (2, 1, 0)).astype(jnp.bfloat16)
    w2t = jnp.transpose(w2, (2, 1, 0)).astype(jnp.bfloat16)
    wst = jnp.transpose(ws, (2, 1, 0))[0].astype(jnp.bfloat16)
    w1b = jnp.concatenate([w1t[1], wst], axis=1)

    def full(shp):
        n = len(shp)
        return pl.BlockSpec(shp, lambda b: (0,) * n)

    xin_spec = pl.BlockSpec((nb, Cin, L), lambda b: (b, 0, 0))
    row_spec = pl.BlockSpec((nb, L, Cout), lambda b: (b, 0, 0))
    st_spec = pl.BlockSpec((1, 8, Cout), lambda b: (b, 0, 0))
    vec_spec = pl.BlockSpec((1, Cout), lambda b: (0, 0))

    xbf, y1, sta = pl.pallas_call(
        _stage_a_kernel,
        grid=(G,),
        in_specs=[xin_spec, full((Cin, 2 * Cout)), full((Cin, Cout)),
                  full((Cin, Cout))],
        out_specs=[xin_spec, row_spec, st_spec],
        out_shape=[jax.ShapeDtypeStruct((N, Cin, L), jnp.bfloat16),
                   jax.ShapeDtypeStruct((N, L, Cout), jnp.bfloat16),
                   jax.ShapeDtypeStruct((G, 8, Cout), jnp.float32)],
        compiler_params=_cparams(),
    )(x, w1b, w1t[0], w1t[2])

    cf_spec = pl.BlockSpec((nb, Cout, L), lambda b: (b, 0, 0))
    stc_spec = pl.BlockSpec((1, Cout, 8), lambda b: (b, 0, 0))
    col_spec = pl.BlockSpec((Cout, 1), lambda b: (0, 0))

    z, stb = pl.pallas_call(
        functools.partial(_stage_b_kernel, count=count),
        grid=(G,),
        in_specs=[row_spec, full((G, 8, Cout)), vec_spec, vec_spec,
                  full((Cout, Cout)), full((Cout, Cout)), full((Cout, Cout))],
        out_specs=[cf_spec, stc_spec],
        out_shape=[jax.ShapeDtypeStruct((N, Cout, L), jnp.bfloat16),
                   jax.ShapeDtypeStruct((G, Cout, 8), jnp.float32)],
        scratch_shapes=[pltpu.VMEM((L + 2, Cout), jnp.float32)],
        compiler_params=_cparams(),
    )(y1, sta, g1.reshape(1, Cout), b1.reshape(1, Cout),
      w2t[0], w2t[1], w2t[2])

    out = pl.pallas_call(
        functools.partial(_stage_c_kernel, count=count),
        grid=(G,),
        in_specs=[cf_spec, xin_spec, full((Cin, Cout)), full((G, Cout, 8)),
                  full((G, 8, Cout)), col_spec, col_spec, col_spec, col_spec],
        out_specs=cf_spec,
        out_shape=jax.ShapeDtypeStruct((N, Cout, L), jnp.float32),
        compiler_params=_cparams(),
    )(z, xbf, wst, stb, sta, g2.reshape(Cout, 1), b2.reshape(Cout, 1),
      gs.reshape(Cout, 1), bs.reshape(Cout, 1))

    return out


def kernel(x, w1, w2, g1, b1, g2, b2, ws, gs, bs):
    return _block(x, w1, w2, g1, b1, g2, b2, ws, gs, bs)
